# baseline (speedup 1.0000x reference)
import functools

import jax
import jax.numpy as jnp
from jax import lax
from jax.experimental import pallas as pl
from jax.experimental.pallas import tpu as pltpu
from jax.experimental.pallas import tpu_sc as plsc

N = 100000
NP = 100352
E = 1600000
B = 4096
NC = 2
NS = 16
CH = 128
ROWS2D = 12544
EP = ROWS2D * CH
TPT = ROWS2D // NS
STG = 16
NSTG = TPT // STG
TRASH = N
STRIPE = 12544
SLAB_ROWS = NS * STRIPE
RB = 14336
RB16 = RB // 16
NB = NP // RB
NU = NP // 16
AU = SLAB_ROWS // 16
HB = 1024


def _leaky(x):
    return jnp.where(x >= 0, x, 0.01 * x)


def _sc_scatter_body(n_g, x_hbm, src_hbm, dst_hbm, typ_hbm, zeros_hbm,
                     agg_hbm, slab, srcb, dstb, typb, gsrc, gdst, rows, sem,
                     sems):
    core = lax.axis_index("c")
    sid = lax.axis_index("s")
    n_pass = n_g // NC

    def do_pass(f, carry):
        g = core * n_pass + f
        gN = g * NP
        pltpu.sync_copy(zeros_hbm, slab.at[pl.ds(sid * STRIPE, STRIPE)])
        plsc.subcore_barrier()

        def do_stage(s, carry2):
            row0 = sid * TPT + s * STG
            pltpu.sync_copy(src_hbm.at[pl.ds(row0, STG)], srcb)
            pltpu.sync_copy(dst_hbm.at[pl.ds(row0, STG)], dstb)
            pltpu.sync_copy(typ_hbm.at[pl.ds(row0, STG)], typb)
            for k in range(STG):
                for i in range(CH // 16):
                    sl = pl.ds(i * 16, 16)
                    gsrc[k, sl] = srcb[k, sl] + gN
                    gdst[k, sl] = dstb[k, sl] + typb[k, sl] * NP
            gcp = [None] * STG
            scp = [None] * STG

            def _scatter(k):
                scp[k] = pltpu.async_copy(
                    rows.at[k % 4], slab.at[gdst.at[k]], sems, add=True)

            for k in range(STG):
                if k >= 4:
                    scp[k - 4].wait()
                gcp[k] = pltpu.async_copy(x_hbm.at[gsrc.at[k]],
                                          rows.at[k % 4], sem)
                if k:
                    gcp[k - 1].wait()
                    _scatter(k - 1)
            gcp[STG - 1].wait()
            _scatter(STG - 1)
            for k in range(STG - 4, STG):
                scp[k].wait()
            return carry2

        lax.fori_loop(0, NSTG, do_stage, 0)
        plsc.subcore_barrier()
        pltpu.sync_copy(
            slab.at[pl.ds(sid * STRIPE, STRIPE)],
            agg_hbm.at[g, pl.ds(sid * STRIPE, STRIPE)])
        return carry

    lax.fori_loop(0, n_pass, do_pass, 0)


def _sc_scatter(n_g):
    mesh = plsc.VectorSubcoreMesh(core_axis_name="c", subcore_axis_name="s")
    return pl.kernel(
        functools.partial(_sc_scatter_body, n_g),
        out_type=jax.ShapeDtypeStruct((n_g, SLAB_ROWS, 8), jnp.float32),
        mesh=mesh,
        compiler_params=pltpu.CompilerParams(use_tc_tiling_on_sc=False),
        scratch_types=[
            pltpu.VMEM_SHARED((SLAB_ROWS, 8), jnp.float32),
            pltpu.VMEM((STG, CH), jnp.int32),
            pltpu.VMEM((STG, CH), jnp.int32),
            pltpu.VMEM((STG, CH), jnp.int32),
            pltpu.VMEM((STG, CH), jnp.int32),
            pltpu.VMEM((STG, CH), jnp.int32),
            pltpu.VMEM((4, CH, 8), jnp.float32),
            pltpu.SemaphoreType.DMA,
            pltpu.SemaphoreType.DMA,
        ],
    )


def _head_gather_body(x_hbm, ha_hbm, out_hbm, idxb, gidx, rows, sem):
    core = lax.axis_index("c")
    sid = lax.axis_index("s")
    w = sid * NC + core
    o = w // 4
    q = w % 4
    krows = B // CH // 4

    row0 = (o // 4) * (B // CH) + q * krows
    pltpu.sync_copy(ha_hbm.at[pl.ds(row0, krows)], idxb)
    gN = (o % 4) * NP
    for k in range(krows):
        for i in range(CH // 16):
            sl = pl.ds(i * 16, 16)
            gidx[k, sl] = idxb[k, sl] + gN
    for k in range(krows):
        pltpu.async_copy(x_hbm.at[gidx.at[k]], rows, sem).wait()
        pltpu.sync_copy(
            rows,
            out_hbm.at[pl.ds(o * B + q * (B // 4) + k * CH, CH),
                       pl.ds(0, 8)])


def _head_gather():
    mesh = plsc.VectorSubcoreMesh(core_axis_name="c", subcore_axis_name="s")
    return pl.kernel(
        _head_gather_body,
        out_type=jax.ShapeDtypeStruct((8 * B, 128), jnp.float32),
        mesh=mesh,
        compiler_params=pltpu.CompilerParams(use_tc_tiling_on_sc=False),
        scratch_types=[
            pltpu.VMEM((B // CH // 4, CH), jnp.int32),
            pltpu.VMEM((B // CH // 4, CH), jnp.int32),
            pltpu.VMEM((CH, 8), jnp.float32),
            pltpu.SemaphoreType.DMA,
        ],
    )


def _unpack(m):
    return jnp.concatenate([m[:, 8 * k:8 * k + 8] for k in range(16)],
                           axis=0)


def _repack(a):
    return jnp.concatenate([a[RB16 * k:RB16 * (k + 1), :] for k in range(16)],
                           axis=1)


def _tc_layer_body(n_g, x_ref, agg0_ref, agg1_ref, wroot_ref, wrel_ref,
                   b_ref, out_ref):
    out = jnp.zeros((RB, 32), jnp.float32) + b_ref[...]
    for g in range(n_g):
        ws = pl.ds(g * 8, 8)
        out += jnp.dot(_unpack(x_ref[g]), wroot_ref[ws, :],
                       preferred_element_type=jnp.float32)
        out += jnp.dot(_unpack(agg0_ref[g]), wrel_ref[0, ws, :],
                       preferred_element_type=jnp.float32)
        out += jnp.dot(_unpack(agg1_ref[g]), wrel_ref[1, ws, :],
                       preferred_element_type=jnp.float32)
    out = _leaky(out)
    for go in range(4):
        out_ref[go] = _repack(out[:, go * 8:(go + 1) * 8])


def _tc_layer(n_g, d_in):
    return pl.pallas_call(
        functools.partial(_tc_layer_body, n_g),
        grid=(NB,),
        in_specs=[
            pl.BlockSpec((n_g, RB16, 128), lambda i: (0, i, 0)),
            pl.BlockSpec((n_g, RB16, 128), lambda i: (0, i, 0)),
            pl.BlockSpec((n_g, RB16, 128), lambda i: (0, i + NB, 0)),
            pl.BlockSpec((d_in, 32), lambda i: (0, 0)),
            pl.BlockSpec((2, d_in, 32), lambda i: (0, 0, 0)),
            pl.BlockSpec((1, 32), lambda i: (0, 0)),
        ],
        out_specs=pl.BlockSpec((4, RB16, 128), lambda i: (0, i, 0)),
        out_shape=jax.ShapeDtypeStruct((4, NU, 128), jnp.float32),
    )


def _tc_head_body(h0_ref, h1_ref, h2_ref, h3_ref, h4_ref, h5_ref, h6_ref,
                  h7_ref, w0_ref, b0_ref, w1_ref, b1_ref, w2_ref, b2_ref,
                  w3_ref, b3_ref, w4_ref, b4_ref, out_ref):
    h_refs = (h0_ref, h1_ref, h2_ref, h3_ref, h4_ref, h5_ref, h6_ref,
              h7_ref)
    h = jnp.zeros((HB, 8), jnp.float32) + b0_ref[...]
    for o in range(8):
        h += jnp.dot(h_refs[o][...][:, 0:8], w0_ref[pl.ds(o * 8, 8), :],
                     preferred_element_type=jnp.float32)
    h = _leaky(h)
    for w_ref, b_ref in ((w1_ref, b1_ref), (w2_ref, b2_ref),
                         (w3_ref, b3_ref)):
        h = _leaky(jnp.dot(h, w_ref[...],
                           preferred_element_type=jnp.float32) + b_ref[...])
    h = _leaky(jnp.dot(h, w4_ref[...],
                       preferred_element_type=jnp.float32) + b4_ref[...])
    m = jnp.max(h, axis=1, keepdims=True)
    h = h - m
    out_ref[...] = h - jnp.log(jnp.sum(jnp.exp(h), axis=1, keepdims=True))


def _tc_head():
    def _hspec(o):
        return pl.BlockSpec((HB, 128), lambda i, o=o: (o * (B // HB) + i, 0))

    return pl.pallas_call(
        _tc_head_body,
        grid=(B // HB,),
        in_specs=[_hspec(0), _hspec(1), _hspec(2), _hspec(3), _hspec(4),
                  _hspec(5), _hspec(6), _hspec(7),
            pl.BlockSpec((64, 8), lambda i: (0, 0)),
            pl.BlockSpec((1, 8), lambda i: (0, 0)),
            pl.BlockSpec((8, 8), lambda i: (0, 0)),
            pl.BlockSpec((1, 8), lambda i: (0, 0)),
            pl.BlockSpec((8, 8), lambda i: (0, 0)),
            pl.BlockSpec((1, 8), lambda i: (0, 0)),
            pl.BlockSpec((8, 8), lambda i: (0, 0)),
            pl.BlockSpec((1, 8), lambda i: (0, 0)),
            pl.BlockSpec((8, 3), lambda i: (0, 0)),
            pl.BlockSpec((1, 3), lambda i: (0, 0)),
        ],
        out_specs=pl.BlockSpec((HB, 3), lambda i: (i, 0)),
        out_shape=jax.ShapeDtypeStruct((B, 3), jnp.float32),
    )


def kernel(edge_index, edge_type, home, away, emb,
           conv0_root, conv0_rel, conv0_b,
           conv1_root, conv1_rel, conv1_b,
           conv2_root, conv2_rel, conv2_b,
           lin0_w, lin0_b, lin1_w, lin1_b, lin2_w, lin2_b,
           lin3_w, lin3_b, lin4_w, lin4_b):
    i32 = jnp.int32
    pe = EP - E
    src2d = jnp.concatenate(
        [edge_index[0].astype(i32), jnp.zeros((pe,), i32)]).reshape(ROWS2D, CH)
    dst2d = jnp.concatenate(
        [edge_index[1].astype(i32),
         jnp.full((pe,), TRASH, i32)]).reshape(ROWS2D, CH)
    typ2d = jnp.concatenate(
        [edge_type.astype(i32), jnp.zeros((pe,), i32)]).reshape(ROWS2D, CH)
    zhbm = jnp.zeros((STRIPE, 8), jnp.float32)

    x = (jnp.pad(emb, ((0, NP - N), (0, 0)))
         .reshape(NP, 2, 8).transpose(1, 0, 2).reshape(2, NU, 128))
    for wroot, wrel, bb in ((conv0_root, conv0_rel, conv0_b),
                            (conv1_root, conv1_rel, conv1_b),
                            (conv2_root, conv2_rel, conv2_b)):
        n_g = x.shape[0]
        agg = _sc_scatter(n_g)(
            x.reshape(n_g * NP, 8), src2d, dst2d, typ2d, zhbm)
        x = _tc_layer(n_g, n_g * 8)(
            x, agg.reshape(n_g, AU, 128), agg.reshape(n_g, AU, 128),
            wroot, wrel, bb.reshape(1, 32))

    h = _head_gather()(
        x.reshape(4 * NP, 8),
        jnp.concatenate([home.astype(i32),
                         away.astype(i32)]).reshape(2 * B // CH, CH))
    out = _tc_head()(
        h, h, h, h, h, h, h, h,
        lin0_w, lin0_b.reshape(1, 8), lin1_w, lin1_b.reshape(1, 8),
        lin2_w, lin2_b.reshape(1, 8), lin3_w, lin3_b.reshape(1, 8),
        lin4_w, lin4_b.reshape(1, 3))
    return out


# speedup vs baseline: 1.0138x; 1.0138x over previous
import functools

import jax
import jax.numpy as jnp
from jax import lax
from jax.experimental import pallas as pl
from jax.experimental.pallas import tpu as pltpu
from jax.experimental.pallas import tpu_sc as plsc

N = 100000
NP = 100352
E = 1600000
B = 4096
NC = 2
NS = 16
CH = 128
ROWS2D = 12544
EP = ROWS2D * CH
TPT = ROWS2D // NS
STG = 16
NSTG = TPT // STG
TRASH = N
STRIPE = 12544
SLAB_ROWS = NS * STRIPE
RB = 2048
RB16 = RB // 16
NB = NP // RB
NU = NP // 16
AU = SLAB_ROWS // 16
HB = 1024


def _leaky(x):
    return jnp.where(x >= 0, x, 0.01 * x)


def _sc_scatter_body(n_g, x_hbm, src_hbm, dst_hbm, typ_hbm, zeros_hbm,
                     agg_hbm, slab, srcb, dstb, typb, gsrc, gdst, rows, sem,
                     sems):
    core = lax.axis_index("c")
    sid = lax.axis_index("s")
    n_pass = n_g // NC

    def do_pass(f, carry):
        g = core * n_pass + f
        gN = g * NP
        pltpu.sync_copy(zeros_hbm, slab.at[pl.ds(sid * STRIPE, STRIPE)])
        plsc.subcore_barrier()

        def do_stage(s, carry2):
            row0 = sid * TPT + s * STG
            pltpu.sync_copy(src_hbm.at[pl.ds(row0, STG)], srcb)
            pltpu.sync_copy(dst_hbm.at[pl.ds(row0, STG)], dstb)
            pltpu.sync_copy(typ_hbm.at[pl.ds(row0, STG)], typb)
            for k in range(STG):
                for i in range(CH // 16):
                    sl = pl.ds(i * 16, 16)
                    gsrc[k, sl] = srcb[k, sl] + gN
                    gdst[k, sl] = dstb[k, sl] + typb[k, sl] * NP
            gcp = [None] * STG
            scp = [None] * STG

            def _scatter(k):
                scp[k] = pltpu.async_copy(
                    rows.at[k % 4], slab.at[gdst.at[k]], sems, add=True)

            for k in range(STG):
                if k >= 4:
                    scp[k - 4].wait()
                gcp[k] = pltpu.async_copy(x_hbm.at[gsrc.at[k]],
                                          rows.at[k % 4], sem)
                if k:
                    gcp[k - 1].wait()
                    _scatter(k - 1)
            gcp[STG - 1].wait()
            _scatter(STG - 1)
            for k in range(STG - 4, STG):
                scp[k].wait()
            return carry2

        lax.fori_loop(0, NSTG, do_stage, 0)
        plsc.subcore_barrier()
        pltpu.sync_copy(
            slab.at[pl.ds(sid * STRIPE, STRIPE)],
            agg_hbm.at[g, pl.ds(sid * STRIPE, STRIPE)])
        return carry

    lax.fori_loop(0, n_pass, do_pass, 0)


def _sc_scatter(n_g):
    mesh = plsc.VectorSubcoreMesh(core_axis_name="c", subcore_axis_name="s")
    return pl.kernel(
        functools.partial(_sc_scatter_body, n_g),
        out_type=jax.ShapeDtypeStruct((n_g, SLAB_ROWS, 8), jnp.float32),
        mesh=mesh,
        compiler_params=pltpu.CompilerParams(use_tc_tiling_on_sc=False),
        scratch_types=[
            pltpu.VMEM_SHARED((SLAB_ROWS, 8), jnp.float32),
            pltpu.VMEM((STG, CH), jnp.int32),
            pltpu.VMEM((STG, CH), jnp.int32),
            pltpu.VMEM((STG, CH), jnp.int32),
            pltpu.VMEM((STG, CH), jnp.int32),
            pltpu.VMEM((STG, CH), jnp.int32),
            pltpu.VMEM((4, CH, 8), jnp.float32),
            pltpu.SemaphoreType.DMA,
            pltpu.SemaphoreType.DMA,
        ],
    )


def _head_gather_body(x_hbm, ha_hbm, out_hbm, idxb, gidx, rows, sem):
    core = lax.axis_index("c")
    sid = lax.axis_index("s")
    w = sid * NC + core
    o = w // 4
    q = w % 4
    krows = B // CH // 4

    row0 = (o // 4) * (B // CH) + q * krows
    pltpu.sync_copy(ha_hbm.at[pl.ds(row0, krows)], idxb)
    gN = (o % 4) * NP
    for k in range(krows):
        for i in range(CH // 16):
            sl = pl.ds(i * 16, 16)
            gidx[k, sl] = idxb[k, sl] + gN
    for k in range(krows):
        pltpu.async_copy(x_hbm.at[gidx.at[k]], rows, sem).wait()
        pltpu.sync_copy(
            rows,
            out_hbm.at[pl.ds(o * B + q * (B // 4) + k * CH, CH),
                       pl.ds(0, 8)])


def _head_gather():
    mesh = plsc.VectorSubcoreMesh(core_axis_name="c", subcore_axis_name="s")
    return pl.kernel(
        _head_gather_body,
        out_type=jax.ShapeDtypeStruct((8 * B, 128), jnp.float32),
        mesh=mesh,
        compiler_params=pltpu.CompilerParams(use_tc_tiling_on_sc=False),
        scratch_types=[
            pltpu.VMEM((B // CH // 4, CH), jnp.int32),
            pltpu.VMEM((B // CH // 4, CH), jnp.int32),
            pltpu.VMEM((CH, 8), jnp.float32),
            pltpu.SemaphoreType.DMA,
        ],
    )


def _unpack(m):
    return jnp.concatenate([m[:, 8 * k:8 * k + 8] for k in range(16)],
                           axis=0)


def _repack(a):
    return jnp.concatenate([a[RB16 * k:RB16 * (k + 1), :] for k in range(16)],
                           axis=1)


def _tc_layer_body(n_g, x_ref, agg0_ref, agg1_ref, wroot_ref, wrel_ref,
                   b_ref, out_ref):
    out = jnp.zeros((RB, 32), jnp.float32) + b_ref[...]
    for g in range(n_g):
        ws = pl.ds(g * 8, 8)
        out += jnp.dot(_unpack(x_ref[g]), wroot_ref[ws, :],
                       preferred_element_type=jnp.float32)
        out += jnp.dot(_unpack(agg0_ref[g]), wrel_ref[0, ws, :],
                       preferred_element_type=jnp.float32)
        out += jnp.dot(_unpack(agg1_ref[g]), wrel_ref[1, ws, :],
                       preferred_element_type=jnp.float32)
    out = _leaky(out)
    for go in range(4):
        out_ref[go] = _repack(out[:, go * 8:(go + 1) * 8])


def _tc_layer(n_g, d_in):
    return pl.pallas_call(
        functools.partial(_tc_layer_body, n_g),
        grid=(NB,),
        in_specs=[
            pl.BlockSpec((n_g, RB16, 128), lambda i: (0, i, 0)),
            pl.BlockSpec((n_g, RB16, 128), lambda i: (0, i, 0)),
            pl.BlockSpec((n_g, RB16, 128), lambda i: (0, i + NB, 0)),
            pl.BlockSpec((d_in, 32), lambda i: (0, 0)),
            pl.BlockSpec((2, d_in, 32), lambda i: (0, 0, 0)),
            pl.BlockSpec((1, 32), lambda i: (0, 0)),
        ],
        out_specs=pl.BlockSpec((4, RB16, 128), lambda i: (0, i, 0)),
        out_shape=jax.ShapeDtypeStruct((4, NU, 128), jnp.float32),
    )


def _tc_head_body(h0_ref, h1_ref, h2_ref, h3_ref, h4_ref, h5_ref, h6_ref,
                  h7_ref, w0_ref, b0_ref, w1_ref, b1_ref, w2_ref, b2_ref,
                  w3_ref, b3_ref, w4_ref, b4_ref, out_ref):
    h_refs = (h0_ref, h1_ref, h2_ref, h3_ref, h4_ref, h5_ref, h6_ref,
              h7_ref)
    h = jnp.zeros((HB, 8), jnp.float32) + b0_ref[...]
    for o in range(8):
        h += jnp.dot(h_refs[o][...][:, 0:8], w0_ref[pl.ds(o * 8, 8), :],
                     preferred_element_type=jnp.float32)
    h = _leaky(h)
    for w_ref, b_ref in ((w1_ref, b1_ref), (w2_ref, b2_ref),
                         (w3_ref, b3_ref)):
        h = _leaky(jnp.dot(h, w_ref[...],
                           preferred_element_type=jnp.float32) + b_ref[...])
    h = _leaky(jnp.dot(h, w4_ref[...],
                       preferred_element_type=jnp.float32) + b4_ref[...])
    m = jnp.max(h, axis=1, keepdims=True)
    h = h - m
    out_ref[...] = h - jnp.log(jnp.sum(jnp.exp(h), axis=1, keepdims=True))


def _tc_head():
    def _hspec(o):
        return pl.BlockSpec((HB, 128), lambda i, o=o: (o * (B // HB) + i, 0))

    return pl.pallas_call(
        _tc_head_body,
        grid=(B // HB,),
        in_specs=[_hspec(0), _hspec(1), _hspec(2), _hspec(3), _hspec(4),
                  _hspec(5), _hspec(6), _hspec(7),
            pl.BlockSpec((64, 8), lambda i: (0, 0)),
            pl.BlockSpec((1, 8), lambda i: (0, 0)),
            pl.BlockSpec((8, 8), lambda i: (0, 0)),
            pl.BlockSpec((1, 8), lambda i: (0, 0)),
            pl.BlockSpec((8, 8), lambda i: (0, 0)),
            pl.BlockSpec((1, 8), lambda i: (0, 0)),
            pl.BlockSpec((8, 8), lambda i: (0, 0)),
            pl.BlockSpec((1, 8), lambda i: (0, 0)),
            pl.BlockSpec((8, 3), lambda i: (0, 0)),
            pl.BlockSpec((1, 3), lambda i: (0, 0)),
        ],
        out_specs=pl.BlockSpec((HB, 3), lambda i: (i, 0)),
        out_shape=jax.ShapeDtypeStruct((B, 3), jnp.float32),
    )


def kernel(edge_index, edge_type, home, away, emb,
           conv0_root, conv0_rel, conv0_b,
           conv1_root, conv1_rel, conv1_b,
           conv2_root, conv2_rel, conv2_b,
           lin0_w, lin0_b, lin1_w, lin1_b, lin2_w, lin2_b,
           lin3_w, lin3_b, lin4_w, lin4_b):
    i32 = jnp.int32
    pe = EP - E
    src2d = jnp.concatenate(
        [edge_index[0].astype(i32), jnp.zeros((pe,), i32)]).reshape(ROWS2D, CH)
    dst2d = jnp.concatenate(
        [edge_index[1].astype(i32),
         jnp.full((pe,), TRASH, i32)]).reshape(ROWS2D, CH)
    typ2d = jnp.concatenate(
        [edge_type.astype(i32), jnp.zeros((pe,), i32)]).reshape(ROWS2D, CH)
    zhbm = jnp.zeros((STRIPE, 8), jnp.float32)

    x = (jnp.pad(emb, ((0, NP - N), (0, 0)))
         .reshape(NP, 2, 8).transpose(1, 0, 2).reshape(2, NU, 128))
    for wroot, wrel, bb in ((conv0_root, conv0_rel, conv0_b),
                            (conv1_root, conv1_rel, conv1_b),
                            (conv2_root, conv2_rel, conv2_b)):
        n_g = x.shape[0]
        agg = _sc_scatter(n_g)(
            x.reshape(n_g * NP, 8), src2d, dst2d, typ2d, zhbm)
        x = _tc_layer(n_g, n_g * 8)(
            x, agg.reshape(n_g, AU, 128), agg.reshape(n_g, AU, 128),
            wroot, wrel, bb.reshape(1, 32))

    h = _head_gather()(
        x.reshape(4 * NP, 8),
        jnp.concatenate([home.astype(i32),
                         away.astype(i32)]).reshape(2 * B // CH, CH))
    out = _tc_head()(
        h, h, h, h, h, h, h, h,
        lin0_w, lin0_b.reshape(1, 8), lin1_w, lin1_b.reshape(1, 8),
        lin2_w, lin2_b.reshape(1, 8), lin3_w, lin3_b.reshape(1, 8),
        lin4_w, lin4_b.reshape(1, 3))
    return out
